# baseline (speedup 1.0000x reference)
import functools

import jax
import jax.numpy as jnp
import numpy as np
from jax import lax
from jax.experimental import pallas as pl
from jax.experimental.pallas import tpu as pltpu
from jax.experimental.pallas import tpu_sc as plsc

DIM = 128
NQ = 64
NW = 32
RPW = 16
NCLS = 13
NREL = 8
NROLE = NCLS + NREL
BATCH = 512
M = 0.1

_C_NF1A, _C_NF1B = 0, 16
_C_NF2A, _C_NF2B, _C_NF2C = 32, 48, 64
_C_NF3A, _C_NF3B = 80, 96
_C_NF4A, _C_NF4B = 112, 128
_C_DISA, _C_DISB = 144, 160
_C_NEGA, _C_NEGB = 176, 192
_R_NF3, _R_NF4 = 0, 16
_R_RIA, _R_RIB = 32, 48
_R_RCA, _R_RCB, _R_RCC = 64, 80, 96
_R_NEG = 112


def _sc_stage1_body(idx_h, cex_h, crad_h, re_h, out_h,
                    idx_v, crad_v, cls_v, rel_v, stage_v, sem):
    w = lax.axis_index("s") * 2 + lax.axis_index("c")
    iota = lax.iota(jnp.int32, 16)
    last = iota == 15

    pltpu.sync_copy(idx_h.at[w], idx_v)
    c0 = pltpu.async_copy(crad_h, crad_v, sem)
    c1 = pltpu.async_copy(cex_h.at[idx_v.at[pl.ds(0, 104)]],
                          cls_v.at[pl.ds(0, 104)], sem)
    c2 = pltpu.async_copy(cex_h.at[idx_v.at[pl.ds(104, 104)]],
                          cls_v.at[pl.ds(104, 104)], sem)
    c3 = pltpu.async_copy(re_h.at[idx_v.at[pl.ds(NCLS * RPW, NREL * RPW)]],
                          rel_v, sem)
    c0.wait(); c1.wait(); c2.wait(); c3.wait()

    zero = jnp.zeros((16,), jnp.float32)

    def put(slot, i, vec):
        s = jnp.full((16,), jnp.sum(vec), jnp.float32)
        idx = jnp.full((16,), slot * RPW, jnp.int32) + i
        plsc.store_scatter(stage_v, [idx], s, mask=last)

    def put_rad(slot, role_off):
        iv = idx_v[pl.ds(role_off, 16)]
        rad = jnp.abs(plsc.load_gather(crad_v, [iv]))
        plsc.store_scatter(stage_v, [iota + slot * RPW], rad)

    def two_class(qb, offa, offb):
        def row(i, _):
            d = zero; n1 = zero; n2 = zero
            for ch in range(8):
                v1 = cls_v[offa + i, pl.ds(ch * 16, 16)]
                v2 = cls_v[offb + i, pl.ds(ch * 16, 16)]
                t = v1 - v2
                d = d + t * t
                n1 = n1 + v1 * v1
                n2 = n2 + v2 * v2
            put(qb + 0, i, d)
            put(qb + 1, i, n1)
            put(qb + 2, i, n2)
            return 0
        lax.fori_loop(0, RPW, row, 0)
        put_rad(qb + 3, offa)
        put_rad(qb + 4, offb)

    two_class(0, _C_NF1A, _C_NF1B)
    two_class(32, _C_DISA, _C_DISB)

    def nf2_row(i, _):
        d21 = zero; d31 = zero; d32 = zero
        n1 = zero; n2 = zero; n3 = zero
        for ch in range(8):
            v1 = cls_v[_C_NF2A + i, pl.ds(ch * 16, 16)]
            v2 = cls_v[_C_NF2B + i, pl.ds(ch * 16, 16)]
            v3 = cls_v[_C_NF2C + i, pl.ds(ch * 16, 16)]
            t21 = v2 - v1; t31 = v3 - v1; t32 = v3 - v2
            d21 = d21 + t21 * t21
            d31 = d31 + t31 * t31
            d32 = d32 + t32 * t32
            n1 = n1 + v1 * v1
            n2 = n2 + v2 * v2
            n3 = n3 + v3 * v3
        put(8, i, d21)
        put(9, i, d31)
        put(10, i, d32)
        put(11, i, n1)
        put(12, i, n2)
        put(13, i, n3)
        return 0
    lax.fori_loop(0, RPW, nf2_row, 0)
    put_rad(14, _C_NF2A)
    put_rad(15, _C_NF2B)

    def rel_between(qb, offc, offd, offr, sign):
        def row(i, _):
            e = zero; n1 = zero; n2 = zero
            for ch in range(8):
                vc = cls_v[offc + i, pl.ds(ch * 16, 16)]
                vr = rel_v[offr + i, pl.ds(ch * 16, 16)]
                vd = cls_v[offd + i, pl.ds(ch * 16, 16)]
                t = (vc + sign * vr) - vd
                e = e + t * t
                n1 = n1 + vc * vc
                n2 = n2 + vd * vd
            put(qb + 0, i, e)
            put(qb + 1, i, n1)
            put(qb + 2, i, n2)
            return 0
        lax.fori_loop(0, RPW, row, 0)
        put_rad(qb + 3, offc)
        put_rad(qb + 4, offd)

    rel_between(16, _C_NF3A, _C_NF3B, _R_NF3, 1.0)
    rel_between(24, _C_NF4A, _C_NF4B, _R_NF4, -1.0)
    rel_between(56, _C_NEGA, _C_NEGB, _R_NEG, 1.0)

    def ri_row(i, _):
        e = zero; dot = zero; m1 = zero; m2 = zero
        for ch in range(8):
            v1 = rel_v[_R_RIA + i, pl.ds(ch * 16, 16)]
            v2 = rel_v[_R_RIB + i, pl.ds(ch * 16, 16)]
            t = v2 - v1
            e = e + t * t
            dot = dot + v1 * v2
            m1 = m1 + v1 * v1
            m2 = m2 + v2 * v2
        put(40, i, e)
        put(41, i, dot)
        put(42, i, m1)
        put(43, i, m2)
        return 0
    lax.fori_loop(0, RPW, ri_row, 0)

    def rch_row(i, _):
        d1 = zero; dot = zero; ncd = zero
        m1 = zero; m2 = zero; m3 = zero
        for ch in range(8):
            v1 = rel_v[_R_RCA + i, pl.ds(ch * 16, 16)]
            v2 = rel_v[_R_RCB + i, pl.ds(ch * 16, 16)]
            v3 = rel_v[_R_RCC + i, pl.ds(ch * 16, 16)]
            t = v1 + v2
            u = v3 - t
            d1 = d1 + u * u
            dot = dot + t * v3
            ncd = ncd + t * t
            m1 = m1 + v1 * v1
            m2 = m2 + v2 * v2
            m3 = m3 + v3 * v3
        put(48, i, d1)
        put(49, i, dot)
        put(50, i, ncd)
        put(51, i, m1)
        put(52, i, m2)
        put(53, i, m3)
        return 0
    lax.fori_loop(0, RPW, rch_row, 0)

    pltpu.sync_copy(stage_v, out_h.at[w])


_sc_stage1 = functools.partial(
    pl.kernel,
    out_type=jax.ShapeDtypeStruct((NW, NQ * RPW), jnp.float32),
    mesh=plsc.VectorSubcoreMesh(core_axis_name="c", subcore_axis_name="s"),
    compiler_params=pltpu.CompilerParams(needs_layout_passes=False),
    scratch_types=[
        pltpu.VMEM((NROLE * RPW,), jnp.int32),
        pltpu.VMEM((1024,), jnp.float32),
        pltpu.VMEM((NCLS * RPW, DIM), jnp.float32),
        pltpu.VMEM((NREL * RPW, DIM), jnp.float32),
        pltpu.VMEM((NQ * RPW,), jnp.float32),
        pltpu.SemaphoreType.DMA,
    ],
)(_sc_stage1_body)


def _tc_finish_body(s_ref, o_ref):
    def q(f, j):
        s = (f * 8 + j) * RPW
        return s_ref[:, pl.ds(s, RPW)]

    sq = jnp.sqrt
    relu = lambda x: jnp.maximum(x, 0.0)
    reg = lambda n2: jnp.abs(sq(n2) - 1.0)
    clipn = lambda n2: jnp.maximum(sq(n2), 1e-12)

    l = relu(sq(q(0, 0)) + q(0, 3) - q(0, 4) - M) + reg(q(0, 1)) + reg(q(0, 2))
    total = jnp.sum(l)
    rc = q(1, 6); rd = q(1, 7); sr = rc + rd
    l = (relu(sq(q(1, 0)) - sr - M) + relu(sq(q(1, 1)) - rc - M)
         + relu(sq(q(1, 2)) - rd - M)
         + reg(q(1, 3)) + reg(q(1, 4)) + reg(q(1, 5)))
    total = total + jnp.sum(l)
    l = relu(sq(q(2, 0)) + q(2, 3) - q(2, 4) - M) + reg(q(2, 1)) + reg(q(2, 2))
    total = total + jnp.sum(l)
    l = (relu(sq(q(3, 0)) - (q(3, 3) + q(3, 4)) - M)
         + reg(q(3, 1)) + reg(q(3, 2)))
    total = total + jnp.sum(l)
    l = (relu((q(4, 3) + q(4, 4)) - sq(q(4, 0)) + M)
         + reg(q(4, 1)) + reg(q(4, 2)))
    total = total + jnp.sum(l)
    direction = q(5, 1) / (clipn(q(5, 2)) * clipn(q(5, 3)))
    l = (relu(sq(q(5, 0)) - M) + reg(q(5, 2)) + reg(q(5, 3))
         + jnp.abs(1.0 - direction))
    total = total + jnp.sum(l)
    direction = q(6, 1) / (clipn(q(6, 2)) * clipn(q(6, 5)))
    l = (relu(sq(q(6, 0)) - M) + reg(q(6, 3)) + reg(q(6, 4)) + reg(q(6, 5))
         + jnp.abs(1.0 - direction))
    total = total + jnp.sum(l)
    l = (-(sq(q(7, 0)) - q(7, 3) - q(7, 4)) + M) + reg(q(7, 1)) + reg(q(7, 2))
    total = total + jnp.sum(l)

    o_ref[0, 0] = total / float(BATCH)


_tc_finish = pl.pallas_call(
    _tc_finish_body,
    out_shape=jax.ShapeDtypeStruct((1, 1), jnp.float32),
    in_specs=[pl.BlockSpec(memory_space=pltpu.VMEM)],
    out_specs=pl.BlockSpec(memory_space=pltpu.SMEM),
)


def _build_perm():
    roles = [(0, 0), (0, 1), (1, 0), (1, 1), (1, 2), (2, 0), (2, 2),
             (3, 1), (3, 2), (4, 0), (4, 1), (7, 0), (7, 2),
             (2, 1), (3, 0), (5, 0), (5, 1), (6, 0), (6, 1), (6, 2), (7, 1)]
    ks = [2, 3, 3, 3, 2, 2, 3, 3]
    offs = np.cumsum([0] + [BATCH * k for k in ks])[:8]
    perm = np.empty((NW, NROLE * RPW), np.int32)
    for w in range(NW):
        for r, (a, j) in enumerate(roles):
            b = w * RPW + np.arange(RPW)
            perm[w, r * RPW:(r + 1) * RPW] = offs[a] + b * ks[a] + j
    return perm


_PERM = _build_perm()


def kernel(nf1, nf2, nf3, nf4, disjoint, role_inclusion, role_chain, nf3_neg,
           class_emb, rel_emb):
    ce = class_emb.astype(jnp.float32)
    cex = ce[:, :DIM]
    crad = jnp.pad(ce[:, DIM], (0, 24))
    re = rel_emb.astype(jnp.float32)

    pool = jnp.concatenate([
        a.astype(jnp.int32).ravel()
        for a in (nf1, nf2, nf3, nf4, disjoint, role_inclusion,
                  role_chain, nf3_neg)])
    idx = pool[jnp.asarray(_PERM)]

    stage = _sc_stage1(idx, cex, crad, re)
    return _tc_finish(stage)[0, 0]


# speedup vs baseline: 1.1773x; 1.1773x over previous
import functools

import jax
import jax.numpy as jnp
import numpy as np
from jax import lax
from jax.experimental import pallas as pl
from jax.experimental.pallas import tpu as pltpu
from jax.experimental.pallas import tpu_sc as plsc

DIM = 128
NQ = 64
NW = 32
RPW = 16
NCLS = 13
NREL = 8
NROLE = NCLS + NREL
BATCH = 512
M = 0.1

_C_NF1A, _C_NF1B = 0, 16
_C_NF2A, _C_NF2B, _C_NF2C = 32, 48, 64
_C_NF3A, _C_NF3B = 80, 96
_C_NF4A, _C_NF4B = 112, 128
_C_DISA, _C_DISB = 144, 160
_C_NEGA, _C_NEGB = 176, 192
_R_NF3, _R_NF4 = 0, 16
_R_RIA, _R_RIB = 32, 48
_R_RCA, _R_RCB, _R_RCC = 64, 80, 96
_R_NEG = 112


def _sc_stage1_body(nf1_h, nf2_h, nf3_h, nf4_h, dis_h, ri_h, rch_h, neg_h,
                    cex_h, crad_h, re_h, out_h,
                    nf1_v, nf2_v, nf3_v, nf4_v, dis_v, ri_v, rch_v, neg_v,
                    idx_v, crad_v, cls_v, rel_v, stage_v, sem):
    w = lax.axis_index("s") * 2 + lax.axis_index("c")
    base = w * RPW
    iota = lax.iota(jnp.int32, 16)
    last = iota == 15

    blocks = [
        pltpu.async_copy(h.at[pl.ds(base, RPW)], v, sem)
        for h, v in ((nf1_h, nf1_v), (nf2_h, nf2_v), (nf3_h, nf3_v),
                     (nf4_h, nf4_v), (dis_h, dis_v), (ri_h, ri_v),
                     (rch_h, rch_v), (neg_h, neg_v))
    ]
    blocks.append(pltpu.async_copy(crad_h, crad_v, sem))
    for c in blocks:
        c.wait()

    rolemap = [(nf1_v, 0), (nf1_v, 1), (nf2_v, 0), (nf2_v, 1), (nf2_v, 2),
               (nf3_v, 0), (nf3_v, 2), (nf4_v, 1), (nf4_v, 2),
               (dis_v, 0), (dis_v, 1), (neg_v, 0), (neg_v, 2),
               (nf3_v, 1), (nf4_v, 0), (ri_v, 0), (ri_v, 1),
               (rch_v, 0), (rch_v, 1), (rch_v, 2), (neg_v, 1)]
    for r, (buf, j) in enumerate(rolemap):
        iv = plsc.load_gather(buf, [iota, jnp.full((16,), j, jnp.int32)])
        idx_v[pl.ds(r * RPW, RPW)] = iv

    c1 = pltpu.async_copy(cex_h.at[idx_v.at[pl.ds(0, 104)]],
                          cls_v.at[pl.ds(0, 104)], sem)
    c2 = pltpu.async_copy(cex_h.at[idx_v.at[pl.ds(104, 104)]],
                          cls_v.at[pl.ds(104, 104)], sem)
    c3 = pltpu.async_copy(re_h.at[idx_v.at[pl.ds(NCLS * RPW, NREL * RPW)]],
                          rel_v, sem)
    c1.wait(); c2.wait(); c3.wait()

    zero = jnp.zeros((16,), jnp.float32)

    def put(slot, i, vec):
        s = jnp.full((16,), jnp.sum(vec), jnp.float32)
        idx = jnp.full((16,), slot * RPW, jnp.int32) + i
        plsc.store_scatter(stage_v, [idx], s, mask=last)

    def put_rad(slot, role_off):
        iv = idx_v[pl.ds(role_off, 16)]
        rad = jnp.abs(plsc.load_gather(crad_v, [iv]))
        plsc.store_scatter(stage_v, [iota + slot * RPW], rad)

    def two_class(qb, offa, offb):
        def row(i, _):
            d = zero; n1 = zero; n2 = zero
            for ch in range(8):
                v1 = cls_v[offa + i, pl.ds(ch * 16, 16)]
                v2 = cls_v[offb + i, pl.ds(ch * 16, 16)]
                t = v1 - v2
                d = d + t * t
                n1 = n1 + v1 * v1
                n2 = n2 + v2 * v2
            put(qb + 0, i, d)
            put(qb + 1, i, n1)
            put(qb + 2, i, n2)
            return 0
        lax.fori_loop(0, RPW, row, 0)
        put_rad(qb + 3, offa)
        put_rad(qb + 4, offb)

    two_class(0, _C_NF1A, _C_NF1B)
    two_class(32, _C_DISA, _C_DISB)

    def nf2_row(i, _):
        d21 = zero; d31 = zero; d32 = zero
        n1 = zero; n2 = zero; n3 = zero
        for ch in range(8):
            v1 = cls_v[_C_NF2A + i, pl.ds(ch * 16, 16)]
            v2 = cls_v[_C_NF2B + i, pl.ds(ch * 16, 16)]
            v3 = cls_v[_C_NF2C + i, pl.ds(ch * 16, 16)]
            t21 = v2 - v1; t31 = v3 - v1; t32 = v3 - v2
            d21 = d21 + t21 * t21
            d31 = d31 + t31 * t31
            d32 = d32 + t32 * t32
            n1 = n1 + v1 * v1
            n2 = n2 + v2 * v2
            n3 = n3 + v3 * v3
        put(8, i, d21)
        put(9, i, d31)
        put(10, i, d32)
        put(11, i, n1)
        put(12, i, n2)
        put(13, i, n3)
        return 0
    lax.fori_loop(0, RPW, nf2_row, 0)
    put_rad(14, _C_NF2A)
    put_rad(15, _C_NF2B)

    def rel_between(qb, offc, offd, offr, sign):
        def row(i, _):
            e = zero; n1 = zero; n2 = zero
            for ch in range(8):
                vc = cls_v[offc + i, pl.ds(ch * 16, 16)]
                vr = rel_v[offr + i, pl.ds(ch * 16, 16)]
                vd = cls_v[offd + i, pl.ds(ch * 16, 16)]
                t = (vc + sign * vr) - vd
                e = e + t * t
                n1 = n1 + vc * vc
                n2 = n2 + vd * vd
            put(qb + 0, i, e)
            put(qb + 1, i, n1)
            put(qb + 2, i, n2)
            return 0
        lax.fori_loop(0, RPW, row, 0)
        put_rad(qb + 3, offc)
        put_rad(qb + 4, offd)

    rel_between(16, _C_NF3A, _C_NF3B, _R_NF3, 1.0)
    rel_between(24, _C_NF4A, _C_NF4B, _R_NF4, -1.0)
    rel_between(56, _C_NEGA, _C_NEGB, _R_NEG, 1.0)

    def ri_row(i, _):
        e = zero; dot = zero; m1 = zero; m2 = zero
        for ch in range(8):
            v1 = rel_v[_R_RIA + i, pl.ds(ch * 16, 16)]
            v2 = rel_v[_R_RIB + i, pl.ds(ch * 16, 16)]
            t = v2 - v1
            e = e + t * t
            dot = dot + v1 * v2
            m1 = m1 + v1 * v1
            m2 = m2 + v2 * v2
        put(40, i, e)
        put(41, i, dot)
        put(42, i, m1)
        put(43, i, m2)
        return 0
    lax.fori_loop(0, RPW, ri_row, 0)

    def rch_row(i, _):
        d1 = zero; dot = zero; ncd = zero
        m1 = zero; m2 = zero; m3 = zero
        for ch in range(8):
            v1 = rel_v[_R_RCA + i, pl.ds(ch * 16, 16)]
            v2 = rel_v[_R_RCB + i, pl.ds(ch * 16, 16)]
            v3 = rel_v[_R_RCC + i, pl.ds(ch * 16, 16)]
            t = v1 + v2
            u = v3 - t
            d1 = d1 + u * u
            dot = dot + t * v3
            ncd = ncd + t * t
            m1 = m1 + v1 * v1
            m2 = m2 + v2 * v2
            m3 = m3 + v3 * v3
        put(48, i, d1)
        put(49, i, dot)
        put(50, i, ncd)
        put(51, i, m1)
        put(52, i, m2)
        put(53, i, m3)
        return 0
    lax.fori_loop(0, RPW, rch_row, 0)

    pltpu.sync_copy(stage_v, out_h.at[w])


_sc_stage1 = functools.partial(
    pl.kernel,
    out_type=jax.ShapeDtypeStruct((NW, NQ * RPW), jnp.float32),
    mesh=plsc.VectorSubcoreMesh(core_axis_name="c", subcore_axis_name="s"),
    compiler_params=pltpu.CompilerParams(needs_layout_passes=False),
    scratch_types=[
        pltpu.VMEM((RPW, 2), jnp.int32),
        pltpu.VMEM((RPW, 3), jnp.int32),
        pltpu.VMEM((RPW, 3), jnp.int32),
        pltpu.VMEM((RPW, 3), jnp.int32),
        pltpu.VMEM((RPW, 2), jnp.int32),
        pltpu.VMEM((RPW, 2), jnp.int32),
        pltpu.VMEM((RPW, 3), jnp.int32),
        pltpu.VMEM((RPW, 3), jnp.int32),
        pltpu.VMEM((NROLE * RPW,), jnp.int32),
        pltpu.VMEM((1024,), jnp.float32),
        pltpu.VMEM((NCLS * RPW, DIM), jnp.float32),
        pltpu.VMEM((NREL * RPW, DIM), jnp.float32),
        pltpu.VMEM((NQ * RPW,), jnp.float32),
        pltpu.SemaphoreType.DMA,
    ],
)(_sc_stage1_body)


def _tc_finish_body(s_ref, o_ref):
    def q(f, j):
        s = (f * 8 + j) * RPW
        return s_ref[:, pl.ds(s, RPW)]

    sq = jnp.sqrt
    relu = lambda x: jnp.maximum(x, 0.0)
    reg = lambda n2: jnp.abs(sq(n2) - 1.0)
    clipn = lambda n2: jnp.maximum(sq(n2), 1e-12)

    l = relu(sq(q(0, 0)) + q(0, 3) - q(0, 4) - M) + reg(q(0, 1)) + reg(q(0, 2))
    total = jnp.sum(l)
    rc = q(1, 6); rd = q(1, 7); sr = rc + rd
    l = (relu(sq(q(1, 0)) - sr - M) + relu(sq(q(1, 1)) - rc - M)
         + relu(sq(q(1, 2)) - rd - M)
         + reg(q(1, 3)) + reg(q(1, 4)) + reg(q(1, 5)))
    total = total + jnp.sum(l)
    l = relu(sq(q(2, 0)) + q(2, 3) - q(2, 4) - M) + reg(q(2, 1)) + reg(q(2, 2))
    total = total + jnp.sum(l)
    l = (relu(sq(q(3, 0)) - (q(3, 3) + q(3, 4)) - M)
         + reg(q(3, 1)) + reg(q(3, 2)))
    total = total + jnp.sum(l)
    l = (relu((q(4, 3) + q(4, 4)) - sq(q(4, 0)) + M)
         + reg(q(4, 1)) + reg(q(4, 2)))
    total = total + jnp.sum(l)
    direction = q(5, 1) / (clipn(q(5, 2)) * clipn(q(5, 3)))
    l = (relu(sq(q(5, 0)) - M) + reg(q(5, 2)) + reg(q(5, 3))
         + jnp.abs(1.0 - direction))
    total = total + jnp.sum(l)
    direction = q(6, 1) / (clipn(q(6, 2)) * clipn(q(6, 5)))
    l = (relu(sq(q(6, 0)) - M) + reg(q(6, 3)) + reg(q(6, 4)) + reg(q(6, 5))
         + jnp.abs(1.0 - direction))
    total = total + jnp.sum(l)
    l = (-(sq(q(7, 0)) - q(7, 3) - q(7, 4)) + M) + reg(q(7, 1)) + reg(q(7, 2))
    total = total + jnp.sum(l)

    o_ref[0, 0] = total / float(BATCH)


_tc_finish = pl.pallas_call(
    _tc_finish_body,
    out_shape=jax.ShapeDtypeStruct((1, 1), jnp.float32),
    in_specs=[pl.BlockSpec(memory_space=pltpu.VMEM)],
    out_specs=pl.BlockSpec(memory_space=pltpu.SMEM),
)


def kernel(nf1, nf2, nf3, nf4, disjoint, role_inclusion, role_chain, nf3_neg,
           class_emb, rel_emb):
    ce = class_emb.astype(jnp.float32)
    cex = ce[:, :DIM]
    crad = jnp.pad(ce[:, DIM], (0, 24))
    re = rel_emb.astype(jnp.float32)
    i32 = lambda a: a.astype(jnp.int32)

    stage = _sc_stage1(i32(nf1), i32(nf2), i32(nf3), i32(nf4), i32(disjoint),
                       i32(role_inclusion), i32(role_chain), i32(nf3_neg),
                       cex, crad, re)
    return _tc_finish(stage)[0, 0]


# speedup vs baseline: 1.4081x; 1.1960x over previous
import functools

import jax
import jax.numpy as jnp
import numpy as np
from jax import lax
from jax.experimental import pallas as pl
from jax.experimental.pallas import tpu as pltpu
from jax.experimental.pallas import tpu_sc as plsc

DIM = 128
NQ = 64
NW = 32
RPW = 16
NCLS = 13
NREL = 8
NROLE = NCLS + NREL
BATCH = 512
M = 0.1

_C_NF1A, _C_NF1B = 0, 16
_C_NF2A, _C_NF2B, _C_NF2C = 32, 48, 64
_C_NF3A, _C_NF3B = 80, 96
_C_NF4A, _C_NF4B = 112, 128
_C_DISA, _C_DISB = 144, 160
_C_NEGA, _C_NEGB = 176, 192
_R_NF3, _R_NF4 = 0, 16
_R_RIA, _R_RIB = 32, 48
_R_RCA, _R_RCB, _R_RCC = 64, 80, 96
_R_NEG = 112


def _sc_stage1_body(cat_h, cex_h, crad_h, re_h, out_h,
                    blk_v, idx_v, crad_v, cls_v, rel_v, stage_v, sem):
    w = lax.axis_index("s") * 2 + lax.axis_index("c")
    base = w * RPW
    iota = lax.iota(jnp.int32, 16)
    last = iota == 15

    cb = pltpu.async_copy(cat_h.at[pl.ds(base, RPW)], blk_v, sem)
    cr = pltpu.async_copy(crad_h, crad_v, sem)
    cb.wait(); cr.wait()

    rolemap = [0, 1, 2, 3, 4, 5, 7, 9, 10, 11, 12, 18, 20,
               6, 8, 13, 14, 15, 16, 17, 19]
    for r, j in enumerate(rolemap):
        iv = plsc.load_gather(blk_v, [iota, jnp.full((16,), j, jnp.int32)])
        idx_v[pl.ds(r * RPW, RPW)] = iv

    c1 = pltpu.async_copy(cex_h.at[idx_v.at[pl.ds(0, 104)]],
                          cls_v.at[pl.ds(0, 104)], sem)
    c2 = pltpu.async_copy(cex_h.at[idx_v.at[pl.ds(104, 104)]],
                          cls_v.at[pl.ds(104, 104)], sem)
    c3 = pltpu.async_copy(re_h.at[idx_v.at[pl.ds(NCLS * RPW, NREL * RPW)]],
                          rel_v, sem)
    c1.wait(); c2.wait(); c3.wait()

    zero = jnp.zeros((16,), jnp.float32)

    def put(slot, i, vec):
        s = jnp.full((16,), jnp.sum(vec), jnp.float32)
        idx = jnp.full((16,), slot * RPW, jnp.int32) + i
        plsc.store_scatter(stage_v, [idx], s, mask=last)

    def put_rad(slot, role_off):
        iv = idx_v[pl.ds(role_off, 16)]
        rad = jnp.abs(plsc.load_gather(crad_v, [iv]))
        plsc.store_scatter(stage_v, [iota + slot * RPW], rad)

    def two_class(qb, offa, offb):
        def row(i, _):
            d = zero; n1 = zero; n2 = zero
            for ch in range(8):
                v1 = cls_v[offa + i, pl.ds(ch * 16, 16)]
                v2 = cls_v[offb + i, pl.ds(ch * 16, 16)]
                t = v1 - v2
                d = d + t * t
                n1 = n1 + v1 * v1
                n2 = n2 + v2 * v2
            put(qb + 0, i, d)
            put(qb + 1, i, n1)
            put(qb + 2, i, n2)
            return 0
        lax.fori_loop(0, RPW, row, 0)
        put_rad(qb + 3, offa)
        put_rad(qb + 4, offb)

    two_class(0, _C_NF1A, _C_NF1B)
    two_class(32, _C_DISA, _C_DISB)

    def nf2_row(i, _):
        d21 = zero; d31 = zero; d32 = zero
        n1 = zero; n2 = zero; n3 = zero
        for ch in range(8):
            v1 = cls_v[_C_NF2A + i, pl.ds(ch * 16, 16)]
            v2 = cls_v[_C_NF2B + i, pl.ds(ch * 16, 16)]
            v3 = cls_v[_C_NF2C + i, pl.ds(ch * 16, 16)]
            t21 = v2 - v1; t31 = v3 - v1; t32 = v3 - v2
            d21 = d21 + t21 * t21
            d31 = d31 + t31 * t31
            d32 = d32 + t32 * t32
            n1 = n1 + v1 * v1
            n2 = n2 + v2 * v2
            n3 = n3 + v3 * v3
        put(8, i, d21)
        put(9, i, d31)
        put(10, i, d32)
        put(11, i, n1)
        put(12, i, n2)
        put(13, i, n3)
        return 0
    lax.fori_loop(0, RPW, nf2_row, 0)
    put_rad(14, _C_NF2A)
    put_rad(15, _C_NF2B)

    def rel_between(qb, offc, offd, offr, sign):
        def row(i, _):
            e = zero; n1 = zero; n2 = zero
            for ch in range(8):
                vc = cls_v[offc + i, pl.ds(ch * 16, 16)]
                vr = rel_v[offr + i, pl.ds(ch * 16, 16)]
                vd = cls_v[offd + i, pl.ds(ch * 16, 16)]
                t = (vc + sign * vr) - vd
                e = e + t * t
                n1 = n1 + vc * vc
                n2 = n2 + vd * vd
            put(qb + 0, i, e)
            put(qb + 1, i, n1)
            put(qb + 2, i, n2)
            return 0
        lax.fori_loop(0, RPW, row, 0)
        put_rad(qb + 3, offc)
        put_rad(qb + 4, offd)

    rel_between(16, _C_NF3A, _C_NF3B, _R_NF3, 1.0)
    rel_between(24, _C_NF4A, _C_NF4B, _R_NF4, -1.0)
    rel_between(56, _C_NEGA, _C_NEGB, _R_NEG, 1.0)

    def ri_row(i, _):
        e = zero; dot = zero; m1 = zero; m2 = zero
        for ch in range(8):
            v1 = rel_v[_R_RIA + i, pl.ds(ch * 16, 16)]
            v2 = rel_v[_R_RIB + i, pl.ds(ch * 16, 16)]
            t = v2 - v1
            e = e + t * t
            dot = dot + v1 * v2
            m1 = m1 + v1 * v1
            m2 = m2 + v2 * v2
        put(40, i, e)
        put(41, i, dot)
        put(42, i, m1)
        put(43, i, m2)
        return 0
    lax.fori_loop(0, RPW, ri_row, 0)

    def rch_row(i, _):
        d1 = zero; dot = zero; ncd = zero
        m1 = zero; m2 = zero; m3 = zero
        for ch in range(8):
            v1 = rel_v[_R_RCA + i, pl.ds(ch * 16, 16)]
            v2 = rel_v[_R_RCB + i, pl.ds(ch * 16, 16)]
            v3 = rel_v[_R_RCC + i, pl.ds(ch * 16, 16)]
            t = v1 + v2
            u = v3 - t
            d1 = d1 + u * u
            dot = dot + t * v3
            ncd = ncd + t * t
            m1 = m1 + v1 * v1
            m2 = m2 + v2 * v2
            m3 = m3 + v3 * v3
        put(48, i, d1)
        put(49, i, dot)
        put(50, i, ncd)
        put(51, i, m1)
        put(52, i, m2)
        put(53, i, m3)
        return 0
    lax.fori_loop(0, RPW, rch_row, 0)

    pltpu.sync_copy(stage_v, out_h.at[w])


_sc_stage1 = functools.partial(
    pl.kernel,
    out_type=jax.ShapeDtypeStruct((NW, NQ * RPW), jnp.float32),
    mesh=plsc.VectorSubcoreMesh(core_axis_name="c", subcore_axis_name="s"),
    compiler_params=pltpu.CompilerParams(needs_layout_passes=False),
    scratch_types=[
        pltpu.VMEM((RPW, NROLE), jnp.int32),
        pltpu.VMEM((NROLE * RPW,), jnp.int32),
        pltpu.VMEM((1024,), jnp.float32),
        pltpu.VMEM((NCLS * RPW, DIM), jnp.float32),
        pltpu.VMEM((NREL * RPW, DIM), jnp.float32),
        pltpu.VMEM((NQ * RPW,), jnp.float32),
        pltpu.SemaphoreType.DMA,
    ],
)(_sc_stage1_body)


def _tc_finish_body(s_ref, o_ref):
    def q(f, j):
        s = (f * 8 + j) * RPW
        return s_ref[:, pl.ds(s, RPW)]

    sq = jnp.sqrt
    relu = lambda x: jnp.maximum(x, 0.0)
    reg = lambda n2: jnp.abs(sq(n2) - 1.0)
    clipn = lambda n2: jnp.maximum(sq(n2), 1e-12)

    l = relu(sq(q(0, 0)) + q(0, 3) - q(0, 4) - M) + reg(q(0, 1)) + reg(q(0, 2))
    total = jnp.sum(l)
    rc = q(1, 6); rd = q(1, 7); sr = rc + rd
    l = (relu(sq(q(1, 0)) - sr - M) + relu(sq(q(1, 1)) - rc - M)
         + relu(sq(q(1, 2)) - rd - M)
         + reg(q(1, 3)) + reg(q(1, 4)) + reg(q(1, 5)))
    total = total + jnp.sum(l)
    l = relu(sq(q(2, 0)) + q(2, 3) - q(2, 4) - M) + reg(q(2, 1)) + reg(q(2, 2))
    total = total + jnp.sum(l)
    l = (relu(sq(q(3, 0)) - (q(3, 3) + q(3, 4)) - M)
         + reg(q(3, 1)) + reg(q(3, 2)))
    total = total + jnp.sum(l)
    l = (relu((q(4, 3) + q(4, 4)) - sq(q(4, 0)) + M)
         + reg(q(4, 1)) + reg(q(4, 2)))
    total = total + jnp.sum(l)
    direction = q(5, 1) / (clipn(q(5, 2)) * clipn(q(5, 3)))
    l = (relu(sq(q(5, 0)) - M) + reg(q(5, 2)) + reg(q(5, 3))
         + jnp.abs(1.0 - direction))
    total = total + jnp.sum(l)
    direction = q(6, 1) / (clipn(q(6, 2)) * clipn(q(6, 5)))
    l = (relu(sq(q(6, 0)) - M) + reg(q(6, 3)) + reg(q(6, 4)) + reg(q(6, 5))
         + jnp.abs(1.0 - direction))
    total = total + jnp.sum(l)
    l = (-(sq(q(7, 0)) - q(7, 3) - q(7, 4)) + M) + reg(q(7, 1)) + reg(q(7, 2))
    total = total + jnp.sum(l)

    o_ref[0, 0] = total / float(BATCH)


_tc_finish = pl.pallas_call(
    _tc_finish_body,
    out_shape=jax.ShapeDtypeStruct((1, 1), jnp.float32),
    in_specs=[pl.BlockSpec(memory_space=pltpu.VMEM)],
    out_specs=pl.BlockSpec(memory_space=pltpu.SMEM),
)


def kernel(nf1, nf2, nf3, nf4, disjoint, role_inclusion, role_chain, nf3_neg,
           class_emb, rel_emb):
    ce = class_emb.astype(jnp.float32)
    cex = ce[:, :DIM]
    crad = jnp.pad(ce[:, DIM], (0, 24))
    re = rel_emb.astype(jnp.float32)
    cat = jnp.concatenate(
        [nf1, nf2, nf3, nf4, disjoint, role_inclusion, role_chain, nf3_neg],
        axis=1).astype(jnp.int32)

    stage = _sc_stage1(cat, cex, crad, re)
    return _tc_finish(stage)[0, 0]
